# Initial kernel scaffold; baseline (speedup 1.0000x reference)
import jax
import jax.numpy as jnp
from jax.experimental import pallas as pl


def kernel(xd, xt, xt_edge_index, xt_batch, y, params):
    raise NotImplementedError("write your pallas kernel here")


# speedup vs baseline: 5.6355x; 5.6355x over previous
import functools

import jax
import jax.numpy as jnp
import numpy as np
from jax import lax
from jax.experimental import pallas as pl
from jax.experimental.pallas import tpu as pltpu
from jax.experimental.pallas import tpu_sc as plsc

_NC = 2
_NS = 16
_LANE = 16

_DIM = 32
_PADROWS = 64
_J = 4


def _sc_mesh():
    return plsc.VectorSubcoreMesh(core_axis_name="c", subcore_axis_name="s",
                                  num_cores=_NC, num_subcores=_NS)


def _make_sc_prep(n, ep, eps, nch, n2):
    ce = _J * 128

    @functools.partial(
        pl.kernel,
        mesh=_sc_mesh(),
        out_type=jax.ShapeDtypeStruct((_NC * ep,), jnp.int32),
        compiler_params=pltpu.CompilerParams(use_tc_tiling_on_sc=False),
        scratch_types=[
            pltpu.VMEM((ce,), jnp.int32),
            pltpu.VMEM((ce,), jnp.int32),
        ],
    )
    def prep(dst_hbm, loc_hbm, dv, lv):
        c = lax.axis_index("c")
        s = lax.axis_index("s")
        lo = c * n2
        base = s * eps

        def chunk(t, carry):
            r0 = pl.multiple_of(base + t * ce, ce)
            pltpu.sync_copy(dst_hbm.at[pl.ds(r0, ce)], dv)
            for q in range(ce // _LANE):
                d = dv[pl.ds(q * _LANE, _LANE)]
                loc = d - lo
                ok = (loc >= 0) & (loc < n2)
                spread = n2 + (
                    (lax.iota(jnp.int32, _LANE) + q * _LANE) & (_PADROWS - 1))
                lv[pl.ds(q * _LANE, _LANE)] = jnp.where(ok, loc, spread)
            pltpu.sync_copy(lv, loc_hbm.at[pl.ds(pl.multiple_of(c * ep + r0, ce), ce)])
            return carry

        lax.fori_loop(0, nch, chunk, 0)

    return prep


def _make_sc_agg(n, ep, eps, nch, n2):
    zr = 128
    accr = -(-(n2 + _PADROWS) // (zr * _NS)) * (zr * _NS)
    nz = accr // (zr * _NS)
    rows_a = -(-n2 // (8 * _NS)) * 8
    rows_last = n2 - (_NS - 1) * rows_a
    assert rows_last > 0 and rows_last % 8 == 0

    @functools.partial(
        pl.kernel,
        mesh=_sc_mesh(),
        out_type=jax.ShapeDtypeStruct((n, _DIM), jnp.float32),
        compiler_params=pltpu.CompilerParams(use_tc_tiling_on_sc=False),
        scratch_types=(
            [pltpu.VMEM_SHARED((accr, _DIM), jnp.float32)]
            + [pltpu.VMEM((128,), jnp.int32) for _ in range(2 * _J)]
            + [pltpu.VMEM((_J, 128, _DIM), jnp.float32),
               pltpu.VMEM((zr, _DIM), jnp.float32),
               pltpu.SemaphoreType.DMA]
        ),
    )
    def agg(g_hbm, src_hbm, loc_hbm, out_hbm, acc, *rest):
        sv = rest[:_J]
        iv = rest[_J:2 * _J]
        rows, zb, sem = rest[2 * _J:]
        c = lax.axis_index("c")
        s = lax.axis_index("s")

        zero = jnp.zeros((_LANE,), jnp.float32)
        for i in range(zr):
            for q in range(_DIM // _LANE):
                zb[i, pl.ds(q * _LANE, _LANE)] = zero
        for t in range(nz):
            pltpu.sync_copy(
                zb, acc.at[pl.ds(pl.multiple_of((s + _NS * t) * zr, zr), zr)])
        plsc.subcore_barrier()

        base = s * eps

        def chunk(t, carry):
            r0 = pl.multiple_of(base + t * (_J * 128), _J * 128)
            l0 = pl.multiple_of(c * ep + r0, _J * 128)
            for j in range(_J):
                pltpu.sync_copy(src_hbm.at[pl.ds(r0 + j * 128, 128)], sv[j])
                pltpu.sync_copy(loc_hbm.at[pl.ds(l0 + j * 128, 128)], iv[j])
            cps = []
            for j in range(_J):
                cps.append(pltpu.async_copy(g_hbm.at[sv[j]], rows.at[j], sem))
            for cp in cps:
                cp.wait()
            for j in range(_J):
                pltpu.sync_copy(rows.at[j], acc.at[iv[j]], add=True)
            return carry

        lax.fori_loop(0, nch, chunk, 0)
        plsc.subcore_barrier()

        arow = pl.multiple_of(s * rows_a, 8)
        orow = pl.multiple_of(c * n2 + s * rows_a, 8)

        @pl.when(s < _NS - 1)
        def _wr():
            pltpu.sync_copy(acc.at[pl.ds(arow, rows_a)],
                            out_hbm.at[pl.ds(orow, rows_a)])

        @pl.when(s == _NS - 1)
        def _wr_last():
            pltpu.sync_copy(acc.at[pl.ds(arow, rows_last)],
                            out_hbm.at[pl.ds(orow, rows_last)])

    return agg


def _tc_g0(xt, w1, n, r):
    grid = n // r

    def body(x_ref, w_ref, o_ref):
        o_ref[...] = jnp.dot(x_ref[...], w_ref[...],
                             preferred_element_type=jnp.float32)

    return pl.pallas_call(
        body,
        grid=(grid,),
        in_specs=[
            pl.BlockSpec((r, xt.shape[1]), lambda i: (i, 0)),
            pl.BlockSpec(w1.shape, lambda i: (0, 0)),
        ],
        out_specs=pl.BlockSpec((r, _DIM), lambda i: (i, 0)),
        out_shape=jax.ShapeDtypeStruct((n, _DIM), jnp.float32),
    )(xt, w1)


def _tc_layer(g, agg, b1, w2, b2, scale, shift, w1n, n, r):
    grid = n // r

    def body(g_ref, a_ref, b1_ref, w2_ref, b2_ref, sc_ref, sh_ref, wn_ref,
             o_ref):
        z = jnp.maximum(g_ref[...] + a_ref[...] + b1_ref[...], 0.0)
        z = jnp.maximum(
            jnp.dot(z, w2_ref[...], preferred_element_type=jnp.float32)
            + b2_ref[...], 0.0)
        h = z * sc_ref[...] + sh_ref[...]
        o_ref[...] = jnp.dot(h, wn_ref[...],
                             preferred_element_type=jnp.float32)

    vec = pl.BlockSpec((1, _DIM), lambda i: (0, 0))
    mat = pl.BlockSpec((_DIM, _DIM), lambda i: (0, 0))
    return pl.pallas_call(
        body,
        grid=(grid,),
        in_specs=[
            pl.BlockSpec((r, _DIM), lambda i: (i, 0)),
            pl.BlockSpec((r, _DIM), lambda i: (i, 0)),
            vec, mat, vec, vec, vec, mat,
        ],
        out_specs=pl.BlockSpec((r, _DIM), lambda i: (i, 0)),
        out_shape=jax.ShapeDtypeStruct((n, _DIM), jnp.float32),
    )(g, agg, b1, w2, b2, scale, shift, w1n)


def _tc_last(g, agg, batch3, b1, w2, b2, scale, shift, b, n, r):
    grid = n // r

    def body(g_ref, a_ref, bt_ref, b1_ref, w2_ref, b2_ref, sc_ref, sh_ref,
             o_ref):
        i = pl.program_id(0)
        z = jnp.maximum(g_ref[...] + a_ref[...] + b1_ref[...], 0.0)
        z = jnp.maximum(
            jnp.dot(z, w2_ref[...], preferred_element_type=jnp.float32)
            + b2_ref[...], 0.0)
        h = z * sc_ref[...] + sh_ref[...]
        bt = bt_ref[...].reshape(1, r)
        ohT = (bt == lax.broadcasted_iota(jnp.int32, (b, 1), 0)).astype(
            jnp.float32)
        part = jnp.dot(ohT, h, preferred_element_type=jnp.float32)

        @pl.when(i == 0)
        def _init():
            o_ref[...] = part

        @pl.when(i > 0)
        def _acc():
            o_ref[...] += part

    vec = pl.BlockSpec((1, _DIM), lambda i: (0, 0))
    mat = pl.BlockSpec((_DIM, _DIM), lambda i: (0, 0))
    return pl.pallas_call(
        body,
        grid=(grid,),
        in_specs=[
            pl.BlockSpec((r, _DIM), lambda i: (i, 0)),
            pl.BlockSpec((r, _DIM), lambda i: (i, 0)),
            pl.BlockSpec((1, 1, r), lambda i: (i, 0, 0)),
            vec, mat, vec, vec, vec,
        ],
        out_specs=pl.BlockSpec((b, _DIM), lambda i: (0, 0)),
        out_shape=jax.ShapeDtypeStruct((b, _DIM), jnp.float32),
    )(g, agg, batch3, b1, w2, b2, scale, shift)


def _tc_build_g(w2d, fcstack, xdf, table, bsz):
    def body(w_ref, f_ref, x_ref, t_ref, gout_ref, emb_ref):
        gout_ref[...] = jnp.dot(w_ref[...], f_ref[...],
                                preferred_element_type=jnp.float32)
        oh = (x_ref[...] == lax.broadcasted_iota(jnp.int32, (1, 65), 1)
              ).astype(jnp.float32)
        emb_ref[...] = jnp.dot(oh, t_ref[...],
                               preferred_element_type=jnp.float32)

    return pl.pallas_call(
        body,
        out_shape=(
            jax.ShapeDtypeStruct((100, 128 * 128), jnp.float32),
            jax.ShapeDtypeStruct((bsz * 100, 128), jnp.float32),
        ),
    )(w2d, fcstack, xdf, table)


def _tc_head(emb_r, gflat, cbexp, fco, xd_b, pooled, xt_w, xt_b,
             w1a, w1b, b1, w2, b2, w3, b3, bsz):

    def body(e_ref, g_ref, cb_ref, fc_ref, xb_ref, p_ref, xw_ref, xtb_ref,
             w1a_ref, w1b_ref, b1_ref, w2_ref, b2_ref, w3_ref, b3_ref, o_ref):
        xd_out = (jnp.dot(e_ref[...], g_ref[...],
                          preferred_element_type=jnp.float32)
                  + jnp.dot(cb_ref[...], fc_ref[...],
                            preferred_element_type=jnp.float32)
                  + xb_ref[...])
        xt_out = jnp.maximum(
            jnp.dot(p_ref[...], xw_ref[...],
                    preferred_element_type=jnp.float32) + xtb_ref[...], 0.0)
        z = jnp.maximum(
            jnp.dot(xd_out, w1a_ref[...], preferred_element_type=jnp.float32)
            + jnp.dot(xt_out, w1b_ref[...], preferred_element_type=jnp.float32)
            + b1_ref[...], 0.0)
        z = jnp.maximum(
            jnp.dot(z, w2_ref[...], preferred_element_type=jnp.float32)
            + b2_ref[...], 0.0)
        o_ref[...] = (jnp.dot(z, w3_ref[...],
                              preferred_element_type=jnp.float32)
                      + b3_ref[...])

    return pl.pallas_call(
        body,
        out_shape=jax.ShapeDtypeStruct((bsz, 1), jnp.float32),
    )(emb_r, gflat, cbexp, fco, xd_b, pooled, xt_w, xt_b,
      w1a, w1b, b1, w2, b2, w3, b3)


def kernel(xd, xt, xt_edge_index, xt_batch, y, params):
    n = xt.shape[0]
    e = xt_edge_index.shape[1]
    bsz = xd.shape[0]
    n2 = n // _NC
    r = 2000
    inv = np.float32(1.0 / np.sqrt(1.0 + 1e-5))

    ce = _NS * _J * 128
    ep = -(-e // ce) * ce
    eps = ep // _NS
    nch = eps // (_J * 128)
    src = xt_edge_index[0]
    dst = xt_edge_index[1]
    src2 = jnp.pad(src, (0, ep - e)).astype(jnp.int32)
    dst2 = jnp.pad(dst, (0, ep - e), constant_values=n).astype(jnp.int32)

    prep = _make_sc_prep(n, ep, eps, nch, n2)
    locidx = prep(dst2)

    gin = params["gin"]
    bn = params["bn"]
    b1s = [gp["b1"].reshape(1, _DIM) for gp in gin]
    b2s = [gp["b2"].reshape(1, _DIM) for gp in gin]
    w2s = [gp["w2"] for gp in gin]
    scales = [(bp["g"] * inv).reshape(1, _DIM) for bp in bn]
    shifts = [bp["b"].reshape(1, _DIM) for bp in bn]

    fco = params["fc1_xd_w"]
    fc3 = fco.reshape(32, 121, 128)
    fcstack = jnp.concatenate(
        [jnp.pad(fc3, ((0, 0), (k, 7 - k), (0, 0))) for k in range(8)],
        axis=0).reshape(256, 128 * 128)
    w2d = jnp.transpose(params["conv_w"], (1, 2, 0)).reshape(100, 256)
    cbexp = jnp.repeat(params["conv_b"], 121).reshape(1, 3872)
    xdf = xd.reshape(bsz * 100, 1).astype(jnp.int32)
    batch3 = xt_batch.reshape(n // r, 1, r).astype(jnp.int32)
    cls_w1 = params["cls_w1"]

    gout, emb = _tc_build_g(w2d, fcstack, xdf, params["emb_xd"], bsz)
    gflat = gout.reshape(12800, 128)
    emb_r = emb.reshape(bsz, 12800)

    sc_agg = _make_sc_agg(n, ep, eps, nch, n2)
    g = _tc_g0(xt, gin[0]["w1"], n, r)
    for l in range(5):
        agg = sc_agg(g, src2, locidx)
        if l < 4:
            g = _tc_layer(g, agg, b1s[l], w2s[l], b2s[l], scales[l],
                          shifts[l], gin[l + 1]["w1"], n, r)
        else:
            pooled = _tc_last(g, agg, batch3, b1s[l], w2s[l], b2s[l],
                              scales[l], shifts[l], bsz, n, r)

    out2 = _tc_head(
        emb_r, gflat, cbexp, fco, params["fc1_xd_b"].reshape(1, 128),
        pooled, params["fc1_xt_w"], params["fc1_xt_b"].reshape(1, 128),
        cls_w1[:128], cls_w1[128:], params["cls_b1"].reshape(1, 1024),
        params["cls_w2"], params["cls_b2"].reshape(1, 256),
        params["cls_w3"], params["cls_b3"].reshape(1, 1), bsz)
    return (out2.reshape(bsz), y)


# speedup vs baseline: 9.7149x; 1.7239x over previous
import functools

import jax
import jax.numpy as jnp
import numpy as np
from jax import lax
from jax.experimental import pallas as pl
from jax.experimental.pallas import tpu as pltpu
from jax.experimental.pallas import tpu_sc as plsc

_NC = 2
_NS = 16
_LANE = 16

_DIM = 32
_PADROWS = 64
_J = 3
_CE = _J * 128
_PAIR = 2 * _CE


def _sc_mesh():
    return plsc.VectorSubcoreMesh(core_axis_name="c", subcore_axis_name="s",
                                  num_cores=_NC, num_subcores=_NS)


def _make_sc_prep(n, ep, eps, nch, n2):
    ce = _PAIR

    @functools.partial(
        pl.kernel,
        mesh=_sc_mesh(),
        out_type=jax.ShapeDtypeStruct((_NC * ep,), jnp.int32),
        compiler_params=pltpu.CompilerParams(use_tc_tiling_on_sc=False),
        scratch_types=[
            pltpu.VMEM((ce,), jnp.int32),
            pltpu.VMEM((ce,), jnp.int32),
        ],
    )
    def prep(dst_hbm, loc_hbm, dv, lv):
        c = lax.axis_index("c")
        s = lax.axis_index("s")
        lo = c * n2
        base = s * eps

        def chunk(t, carry):
            r0 = pl.multiple_of(base + t * ce, ce)
            pltpu.sync_copy(dst_hbm.at[pl.ds(r0, ce)], dv)
            for q in range(ce // _LANE):
                d = dv[pl.ds(q * _LANE, _LANE)]
                loc = d - lo
                ok = (loc >= 0) & (loc < n2)
                spread = n2 + (
                    (lax.iota(jnp.int32, _LANE) + q * _LANE) & (_PADROWS - 1))
                lv[pl.ds(q * _LANE, _LANE)] = jnp.where(ok, loc, spread)
            pltpu.sync_copy(lv, loc_hbm.at[pl.ds(pl.multiple_of(c * ep + r0, ce), ce)])
            return carry

        lax.fori_loop(0, nch, chunk, 0)

    return prep


def _make_sc_agg(n, ep, eps, nch, n2):
    zs = 112
    accr = -(-(n2 + _PADROWS) // (zs * _NS)) * (zs * _NS)
    nz = accr // (zs * _NS)
    rows_a = -(-n2 // (8 * _NS)) * 8
    rows_last = n2 - (_NS - 1) * rows_a
    assert rows_last > 0 and rows_last % 8 == 0

    @functools.partial(
        pl.kernel,
        mesh=_sc_mesh(),
        out_type=jax.ShapeDtypeStruct((n, _DIM), jnp.float32),
        compiler_params=pltpu.CompilerParams(use_tc_tiling_on_sc=False),
        scratch_types=(
            [pltpu.VMEM_SHARED((accr, _DIM), jnp.float32)]
            + [pltpu.VMEM((128,), jnp.int32) for _ in range(4 * _J)]
            + [pltpu.VMEM((_J, 128, _DIM), jnp.float32),
               pltpu.VMEM((_J, 128, _DIM), jnp.float32),
               pltpu.SemaphoreType.DMA,
               pltpu.SemaphoreType.DMA,
               pltpu.SemaphoreType.DMA]
        ),
    )
    def agg(g_hbm, src_hbm, loc_hbm, out_hbm, acc, *rest):
        sva = rest[:_J]
        svb = rest[_J:2 * _J]
        iva = rest[2 * _J:3 * _J]
        ivb = rest[3 * _J:4 * _J]
        rowsa, rowsb, semi, semg, sems = rest[4 * _J:]
        c = lax.axis_index("c")
        s = lax.axis_index("s")

        zero = jnp.zeros((_LANE,), jnp.float32)
        for j in range(_J):
            for i in range(128):
                for q in range(_DIM // _LANE):
                    rowsa[j, i, pl.ds(q * _LANE, _LANE)] = zero
        zsrc = rowsa.at[0].at[pl.ds(0, zs)]
        for t in range(nz):
            pltpu.sync_copy(
                zsrc, acc.at[pl.ds(pl.multiple_of((s * nz + t) * zs, 16), zs)])
        plsc.subcore_barrier()

        base = s * eps

        def chunk(t, carry):
            e0 = pl.multiple_of(base + t * _PAIR, _PAIR)
            l0 = pl.multiple_of(c * ep + e0, _PAIR)
            ci = []
            for j in range(_J):
                ci.append(pltpu.async_copy(
                    src_hbm.at[pl.ds(e0 + j * 128, 128)], sva[j], semi))
                ci.append(pltpu.async_copy(
                    loc_hbm.at[pl.ds(l0 + j * 128, 128)], iva[j], semi))
                ci.append(pltpu.async_copy(
                    src_hbm.at[pl.ds(e0 + _CE + j * 128, 128)], svb[j], semi))
                ci.append(pltpu.async_copy(
                    loc_hbm.at[pl.ds(l0 + _CE + j * 128, 128)], ivb[j], semi))
            for cp in ci:
                cp.wait()
            ga = [pltpu.async_copy(g_hbm.at[sva[j]], rowsa.at[j], semg)
                  for j in range(_J)]
            for cp in ga:
                cp.wait()
            sa = [pltpu.async_copy(rowsa.at[j], acc.at[iva[j]], sems, add=True)
                  for j in range(_J)]
            gb = [pltpu.async_copy(g_hbm.at[svb[j]], rowsb.at[j], semg)
                  for j in range(_J)]
            for cp in gb:
                cp.wait()
            for cp in sa:
                cp.wait()
            sb = [pltpu.async_copy(rowsb.at[j], acc.at[ivb[j]], sems, add=True)
                  for j in range(_J)]
            for cp in sb:
                cp.wait()
            return carry

        lax.fori_loop(0, nch, chunk, 0)
        plsc.subcore_barrier()

        arow = pl.multiple_of(s * rows_a, 8)
        orow = pl.multiple_of(c * n2 + s * rows_a, 8)

        @pl.when(s < _NS - 1)
        def _wr():
            pltpu.sync_copy(acc.at[pl.ds(arow, rows_a)],
                            out_hbm.at[pl.ds(orow, rows_a)])

        @pl.when(s == _NS - 1)
        def _wr_last():
            pltpu.sync_copy(acc.at[pl.ds(arow, rows_last)],
                            out_hbm.at[pl.ds(orow, rows_last)])

    return agg


def _tc_g0(xt, w1, n, r):
    grid = n // r

    def body(x_ref, w_ref, o_ref):
        o_ref[...] = jnp.dot(x_ref[...], w_ref[...],
                             preferred_element_type=jnp.float32)

    return pl.pallas_call(
        body,
        grid=(grid,),
        in_specs=[
            pl.BlockSpec((r, xt.shape[1]), lambda i: (i, 0)),
            pl.BlockSpec(w1.shape, lambda i: (0, 0)),
        ],
        out_specs=pl.BlockSpec((r, _DIM), lambda i: (i, 0)),
        out_shape=jax.ShapeDtypeStruct((n, _DIM), jnp.float32),
    )(xt, w1)


def _tc_layer(g, agg, b1, w2, b2, scale, shift, w1n, n, r):
    grid = n // r

    def body(g_ref, a_ref, b1_ref, w2_ref, b2_ref, sc_ref, sh_ref, wn_ref,
             o_ref):
        z = jnp.maximum(g_ref[...] + a_ref[...] + b1_ref[...], 0.0)
        z = jnp.maximum(
            jnp.dot(z, w2_ref[...], preferred_element_type=jnp.float32)
            + b2_ref[...], 0.0)
        h = z * sc_ref[...] + sh_ref[...]
        o_ref[...] = jnp.dot(h, wn_ref[...],
                             preferred_element_type=jnp.float32)

    vec = pl.BlockSpec((1, _DIM), lambda i: (0, 0))
    mat = pl.BlockSpec((_DIM, _DIM), lambda i: (0, 0))
    return pl.pallas_call(
        body,
        grid=(grid,),
        in_specs=[
            pl.BlockSpec((r, _DIM), lambda i: (i, 0)),
            pl.BlockSpec((r, _DIM), lambda i: (i, 0)),
            vec, mat, vec, vec, vec, mat,
        ],
        out_specs=pl.BlockSpec((r, _DIM), lambda i: (i, 0)),
        out_shape=jax.ShapeDtypeStruct((n, _DIM), jnp.float32),
    )(g, agg, b1, w2, b2, scale, shift, w1n)


def _tc_last(g, agg, batch3, b1, w2, b2, scale, shift, b, n, r):
    grid = n // r

    def body(g_ref, a_ref, bt_ref, b1_ref, w2_ref, b2_ref, sc_ref, sh_ref,
             o_ref):
        i = pl.program_id(0)
        z = jnp.maximum(g_ref[...] + a_ref[...] + b1_ref[...], 0.0)
        z = jnp.maximum(
            jnp.dot(z, w2_ref[...], preferred_element_type=jnp.float32)
            + b2_ref[...], 0.0)
        h = z * sc_ref[...] + sh_ref[...]
        bt = bt_ref[...].reshape(1, r)
        ohT = (bt == lax.broadcasted_iota(jnp.int32, (b, 1), 0)).astype(
            jnp.float32)
        part = jnp.dot(ohT, h, preferred_element_type=jnp.float32)

        @pl.when(i == 0)
        def _init():
            o_ref[...] = part

        @pl.when(i > 0)
        def _acc():
            o_ref[...] += part

    vec = pl.BlockSpec((1, _DIM), lambda i: (0, 0))
    mat = pl.BlockSpec((_DIM, _DIM), lambda i: (0, 0))
    return pl.pallas_call(
        body,
        grid=(grid,),
        in_specs=[
            pl.BlockSpec((r, _DIM), lambda i: (i, 0)),
            pl.BlockSpec((r, _DIM), lambda i: (i, 0)),
            pl.BlockSpec((1, 1, r), lambda i: (i, 0, 0)),
            vec, mat, vec, vec, vec,
        ],
        out_specs=pl.BlockSpec((b, _DIM), lambda i: (0, 0)),
        out_shape=jax.ShapeDtypeStruct((b, _DIM), jnp.float32),
    )(g, agg, batch3, b1, w2, b2, scale, shift)


def _tc_build_g(w2d, fcstack, xdf, table, bsz):
    def body(w_ref, f_ref, x_ref, t_ref, gout_ref, emb_ref):
        gout_ref[...] = jnp.dot(w_ref[...], f_ref[...],
                                preferred_element_type=jnp.float32)
        oh = (x_ref[...] == lax.broadcasted_iota(jnp.int32, (1, 65), 1)
              ).astype(jnp.float32)
        emb_ref[...] = jnp.dot(oh, t_ref[...],
                               preferred_element_type=jnp.float32)

    return pl.pallas_call(
        body,
        out_shape=(
            jax.ShapeDtypeStruct((100, 128 * 128), jnp.float32),
            jax.ShapeDtypeStruct((bsz * 100, 128), jnp.float32),
        ),
    )(w2d, fcstack, xdf, table)


def _tc_head(emb_r, gflat, cbexp, fco, xd_b, pooled, xt_w, xt_b,
             w1a, w1b, b1, w2, b2, w3, b3, bsz):

    def body(e_ref, g_ref, cb_ref, fc_ref, xb_ref, p_ref, xw_ref, xtb_ref,
             w1a_ref, w1b_ref, b1_ref, w2_ref, b2_ref, w3_ref, b3_ref, o_ref):
        xd_out = (jnp.dot(e_ref[...], g_ref[...],
                          preferred_element_type=jnp.float32)
                  + jnp.dot(cb_ref[...], fc_ref[...],
                            preferred_element_type=jnp.float32)
                  + xb_ref[...])
        xt_out = jnp.maximum(
            jnp.dot(p_ref[...], xw_ref[...],
                    preferred_element_type=jnp.float32) + xtb_ref[...], 0.0)
        z = jnp.maximum(
            jnp.dot(xd_out, w1a_ref[...], preferred_element_type=jnp.float32)
            + jnp.dot(xt_out, w1b_ref[...], preferred_element_type=jnp.float32)
            + b1_ref[...], 0.0)
        z = jnp.maximum(
            jnp.dot(z, w2_ref[...], preferred_element_type=jnp.float32)
            + b2_ref[...], 0.0)
        o_ref[...] = (jnp.dot(z, w3_ref[...],
                              preferred_element_type=jnp.float32)
                      + b3_ref[...])

    return pl.pallas_call(
        body,
        out_shape=jax.ShapeDtypeStruct((bsz, 1), jnp.float32),
    )(emb_r, gflat, cbexp, fco, xd_b, pooled, xt_w, xt_b,
      w1a, w1b, b1, w2, b2, w3, b3)


def kernel(xd, xt, xt_edge_index, xt_batch, y, params):
    n = xt.shape[0]
    e = xt_edge_index.shape[1]
    bsz = xd.shape[0]
    n2 = n // _NC
    r = 2000
    inv = np.float32(1.0 / np.sqrt(1.0 + 1e-5))

    ce = _NS * _PAIR
    ep = -(-e // ce) * ce
    eps = ep // _NS
    nch = eps // _PAIR
    src = xt_edge_index[0]
    dst = xt_edge_index[1]
    src2 = jnp.pad(src, (0, ep - e)).astype(jnp.int32)
    dst2 = jnp.pad(dst, (0, ep - e), constant_values=n).astype(jnp.int32)

    prep = _make_sc_prep(n, ep, eps, nch, n2)
    locidx = prep(dst2)

    gin = params["gin"]
    bn = params["bn"]
    b1s = [gp["b1"].reshape(1, _DIM) for gp in gin]
    b2s = [gp["b2"].reshape(1, _DIM) for gp in gin]
    w2s = [gp["w2"] for gp in gin]
    scales = [(bp["g"] * inv).reshape(1, _DIM) for bp in bn]
    shifts = [bp["b"].reshape(1, _DIM) for bp in bn]

    fco = params["fc1_xd_w"]
    fc3 = fco.reshape(32, 121, 128)
    fcstack = jnp.concatenate(
        [jnp.pad(fc3, ((0, 0), (k, 7 - k), (0, 0))) for k in range(8)],
        axis=0).reshape(256, 128 * 128)
    w2d = jnp.transpose(params["conv_w"], (1, 2, 0)).reshape(100, 256)
    cbexp = jnp.repeat(params["conv_b"], 121).reshape(1, 3872)
    xdf = xd.reshape(bsz * 100, 1).astype(jnp.int32)
    batch3 = xt_batch.reshape(n // r, 1, r).astype(jnp.int32)
    cls_w1 = params["cls_w1"]

    gout, emb = _tc_build_g(w2d, fcstack, xdf, params["emb_xd"], bsz)
    gflat = gout.reshape(12800, 128)
    emb_r = emb.reshape(bsz, 12800)

    sc_agg = _make_sc_agg(n, ep, eps, nch, n2)
    g = _tc_g0(xt, gin[0]["w1"], n, r)
    for l in range(5):
        agg = sc_agg(g, src2, locidx)
        if l < 4:
            g = _tc_layer(g, agg, b1s[l], w2s[l], b2s[l], scales[l],
                          shifts[l], gin[l + 1]["w1"], n, r)
        else:
            pooled = _tc_last(g, agg, batch3, b1s[l], w2s[l], b2s[l],
                              scales[l], shifts[l], bsz, n, r)

    out2 = _tc_head(
        emb_r, gflat, cbexp, fco, params["fc1_xd_b"].reshape(1, 128),
        pooled, params["fc1_xt_w"], params["fc1_xt_b"].reshape(1, 128),
        cls_w1[:128], cls_w1[128:], params["cls_b1"].reshape(1, 1024),
        params["cls_w2"], params["cls_b2"].reshape(1, 256),
        params["cls_w3"], params["cls_b3"].reshape(1, 1), bsz)
    return (out2.reshape(bsz), y)


# speedup vs baseline: 10.5999x; 1.0911x over previous
import functools

import jax
import jax.numpy as jnp
import numpy as np
from jax import lax
from jax.experimental import pallas as pl
from jax.experimental.pallas import tpu as pltpu
from jax.experimental.pallas import tpu_sc as plsc

_NC = 2
_NS = 16
_LANE = 16

_DIM = 32
_PADROWS = 64
_J = 3
_CE = _J * 128
_PAIR = 2 * _CE


def _sc_mesh():
    return plsc.VectorSubcoreMesh(core_axis_name="c", subcore_axis_name="s",
                                  num_cores=_NC, num_subcores=_NS)


def _make_sc_prep(n, ep, eps, nch, n2):
    ce = _PAIR

    @functools.partial(
        pl.kernel,
        mesh=_sc_mesh(),
        out_type=jax.ShapeDtypeStruct((_NC * ep,), jnp.int32),
        compiler_params=pltpu.CompilerParams(use_tc_tiling_on_sc=False),
        scratch_types=[
            pltpu.VMEM((ce,), jnp.int32),
            pltpu.VMEM((ce,), jnp.int32),
        ],
    )
    def prep(dst_hbm, loc_hbm, dv, lv):
        c = lax.axis_index("c")
        s = lax.axis_index("s")
        lo = c * n2
        base = s * eps

        def chunk(t, carry):
            r0 = pl.multiple_of(base + t * ce, ce)
            pltpu.sync_copy(dst_hbm.at[pl.ds(r0, ce)], dv)
            for q in range(ce // _LANE):
                d = dv[pl.ds(q * _LANE, _LANE)]
                loc = d - lo
                ok = (loc >= 0) & (loc < n2)
                spread = n2 + (
                    (lax.iota(jnp.int32, _LANE) + q * _LANE) & (_PADROWS - 1))
                lv[pl.ds(q * _LANE, _LANE)] = jnp.where(ok, loc, spread)
            pltpu.sync_copy(lv, loc_hbm.at[pl.ds(pl.multiple_of(c * ep + r0, ce), ce)])
            return carry

        lax.fori_loop(0, nch, chunk, 0)

    return prep


def _make_sc_agg(n, ep, eps, nch, n2):
    zs = 112
    accr = -(-(n2 + _PADROWS) // (zs * _NS)) * (zs * _NS)
    nz = accr // (zs * _NS)
    rows_a = -(-n2 // (8 * _NS)) * 8
    rows_last = n2 - (_NS - 1) * rows_a
    assert rows_last > 0 and rows_last % 8 == 0

    @functools.partial(
        pl.kernel,
        mesh=_sc_mesh(),
        out_type=jax.ShapeDtypeStruct((n, _DIM), jnp.float32),
        compiler_params=pltpu.CompilerParams(use_tc_tiling_on_sc=False),
        scratch_types=(
            [pltpu.VMEM_SHARED((accr, _DIM), jnp.float32)]
            + [pltpu.VMEM((128,), jnp.int32) for _ in range(4 * _J)]
            + [pltpu.VMEM((_J, 128, _DIM), jnp.float32),
               pltpu.VMEM((_J, 128, _DIM), jnp.float32),
               pltpu.SemaphoreType.DMA,
               pltpu.SemaphoreType.DMA,
               pltpu.SemaphoreType.DMA]
        ),
    )
    def agg(g_hbm, src_hbm, loc_hbm, out_hbm, acc, *rest):
        sva = rest[:_J]
        svb = rest[_J:2 * _J]
        iva = rest[2 * _J:3 * _J]
        ivb = rest[3 * _J:4 * _J]
        rowsa, rowsb, semi, semg, sems = rest[4 * _J:]
        c = lax.axis_index("c")
        s = lax.axis_index("s")

        zero = jnp.zeros((_LANE,), jnp.float32)
        for j in range(_J):
            for i in range(128):
                for q in range(_DIM // _LANE):
                    rowsa[j, i, pl.ds(q * _LANE, _LANE)] = zero
        zsrc = rowsa.at[0].at[pl.ds(0, zs)]
        for t in range(nz):
            pltpu.sync_copy(
                zsrc, acc.at[pl.ds(pl.multiple_of((s * nz + t) * zs, 16), zs)])
        plsc.subcore_barrier()

        base = s * eps

        def chunk(t, carry):
            e0 = pl.multiple_of(base + t * _PAIR, _PAIR)
            l0 = pl.multiple_of(c * ep + e0, _PAIR)
            ci = []
            for j in range(_J):
                ci.append(pltpu.async_copy(
                    src_hbm.at[pl.ds(e0 + j * 128, 128)], sva[j], semi))
                ci.append(pltpu.async_copy(
                    loc_hbm.at[pl.ds(l0 + j * 128, 128)], iva[j], semi))
                ci.append(pltpu.async_copy(
                    src_hbm.at[pl.ds(e0 + _CE + j * 128, 128)], svb[j], semi))
                ci.append(pltpu.async_copy(
                    loc_hbm.at[pl.ds(l0 + _CE + j * 128, 128)], ivb[j], semi))
            for cp in ci:
                cp.wait()
            ga = [pltpu.async_copy(g_hbm.at[sva[j]], rowsa.at[j], semg)
                  for j in range(_J)]
            gb = [pltpu.async_copy(g_hbm.at[svb[j]], rowsb.at[j], semg)
                  for j in range(_J)]
            for cp in ga:
                cp.wait()
            sa = [pltpu.async_copy(rowsa.at[j], acc.at[iva[j]], sems, add=True)
                  for j in range(_J)]
            for cp in gb:
                cp.wait()
            sb = [pltpu.async_copy(rowsb.at[j], acc.at[ivb[j]], sems, add=True)
                  for j in range(_J)]
            for cp in sa:
                cp.wait()
            for cp in sb:
                cp.wait()
            return carry

        lax.fori_loop(0, nch, chunk, 0)
        plsc.subcore_barrier()

        arow = pl.multiple_of(s * rows_a, 8)
        orow = pl.multiple_of(c * n2 + s * rows_a, 8)

        @pl.when(s < _NS - 1)
        def _wr():
            pltpu.sync_copy(acc.at[pl.ds(arow, rows_a)],
                            out_hbm.at[pl.ds(orow, rows_a)])

        @pl.when(s == _NS - 1)
        def _wr_last():
            pltpu.sync_copy(acc.at[pl.ds(arow, rows_last)],
                            out_hbm.at[pl.ds(orow, rows_last)])

    return agg


def _tc_g0(xt, w1, n, r):
    grid = n // r

    def body(x_ref, w_ref, o_ref):
        o_ref[...] = jnp.dot(x_ref[...], w_ref[...],
                             preferred_element_type=jnp.float32)

    return pl.pallas_call(
        body,
        grid=(grid,),
        in_specs=[
            pl.BlockSpec((r, xt.shape[1]), lambda i: (i, 0)),
            pl.BlockSpec(w1.shape, lambda i: (0, 0)),
        ],
        out_specs=pl.BlockSpec((r, _DIM), lambda i: (i, 0)),
        out_shape=jax.ShapeDtypeStruct((n, _DIM), jnp.float32),
    )(xt, w1)


def _tc_layer(g, agg, b1, w2, b2, scale, shift, w1n, n, r):
    grid = n // r

    def body(g_ref, a_ref, b1_ref, w2_ref, b2_ref, sc_ref, sh_ref, wn_ref,
             o_ref):
        z = jnp.maximum(g_ref[...] + a_ref[...] + b1_ref[...], 0.0)
        z = jnp.maximum(
            jnp.dot(z, w2_ref[...], preferred_element_type=jnp.float32)
            + b2_ref[...], 0.0)
        h = z * sc_ref[...] + sh_ref[...]
        o_ref[...] = jnp.dot(h, wn_ref[...],
                             preferred_element_type=jnp.float32)

    vec = pl.BlockSpec((1, _DIM), lambda i: (0, 0))
    mat = pl.BlockSpec((_DIM, _DIM), lambda i: (0, 0))
    return pl.pallas_call(
        body,
        grid=(grid,),
        in_specs=[
            pl.BlockSpec((r, _DIM), lambda i: (i, 0)),
            pl.BlockSpec((r, _DIM), lambda i: (i, 0)),
            vec, mat, vec, vec, vec, mat,
        ],
        out_specs=pl.BlockSpec((r, _DIM), lambda i: (i, 0)),
        out_shape=jax.ShapeDtypeStruct((n, _DIM), jnp.float32),
    )(g, agg, b1, w2, b2, scale, shift, w1n)


def _tc_last(g, agg, batch3, b1, w2, b2, scale, shift, b, n, r):
    grid = n // r

    def body(g_ref, a_ref, bt_ref, b1_ref, w2_ref, b2_ref, sc_ref, sh_ref,
             o_ref):
        i = pl.program_id(0)
        z = jnp.maximum(g_ref[...] + a_ref[...] + b1_ref[...], 0.0)
        z = jnp.maximum(
            jnp.dot(z, w2_ref[...], preferred_element_type=jnp.float32)
            + b2_ref[...], 0.0)
        h = z * sc_ref[...] + sh_ref[...]
        bt = bt_ref[...].reshape(1, r)
        ohT = (bt == lax.broadcasted_iota(jnp.int32, (b, 1), 0)).astype(
            jnp.float32)
        part = jnp.dot(ohT, h, preferred_element_type=jnp.float32)

        @pl.when(i == 0)
        def _init():
            o_ref[...] = part

        @pl.when(i > 0)
        def _acc():
            o_ref[...] += part

    vec = pl.BlockSpec((1, _DIM), lambda i: (0, 0))
    mat = pl.BlockSpec((_DIM, _DIM), lambda i: (0, 0))
    return pl.pallas_call(
        body,
        grid=(grid,),
        in_specs=[
            pl.BlockSpec((r, _DIM), lambda i: (i, 0)),
            pl.BlockSpec((r, _DIM), lambda i: (i, 0)),
            pl.BlockSpec((1, 1, r), lambda i: (i, 0, 0)),
            vec, mat, vec, vec, vec,
        ],
        out_specs=pl.BlockSpec((b, _DIM), lambda i: (0, 0)),
        out_shape=jax.ShapeDtypeStruct((b, _DIM), jnp.float32),
    )(g, agg, batch3, b1, w2, b2, scale, shift)


def _tc_build_g(w2d, fcstack, xdf, table, bsz):
    def body(w_ref, f_ref, x_ref, t_ref, gout_ref, emb_ref):
        gout_ref[...] = jnp.dot(w_ref[...], f_ref[...],
                                preferred_element_type=jnp.float32)
        oh = (x_ref[...] == lax.broadcasted_iota(jnp.int32, (1, 65), 1)
              ).astype(jnp.float32)
        emb_ref[...] = jnp.dot(oh, t_ref[...],
                               preferred_element_type=jnp.float32)

    return pl.pallas_call(
        body,
        out_shape=(
            jax.ShapeDtypeStruct((100, 128 * 128), jnp.float32),
            jax.ShapeDtypeStruct((bsz * 100, 128), jnp.float32),
        ),
    )(w2d, fcstack, xdf, table)


def _tc_head(emb_r, gflat, cbexp, fco, xd_b, pooled, xt_w, xt_b,
             w1a, w1b, b1, w2, b2, w3, b3, bsz):

    def body(e_ref, g_ref, cb_ref, fc_ref, xb_ref, p_ref, xw_ref, xtb_ref,
             w1a_ref, w1b_ref, b1_ref, w2_ref, b2_ref, w3_ref, b3_ref, o_ref):
        xd_out = (jnp.dot(e_ref[...], g_ref[...],
                          preferred_element_type=jnp.float32)
                  + jnp.dot(cb_ref[...], fc_ref[...],
                            preferred_element_type=jnp.float32)
                  + xb_ref[...])
        xt_out = jnp.maximum(
            jnp.dot(p_ref[...], xw_ref[...],
                    preferred_element_type=jnp.float32) + xtb_ref[...], 0.0)
        z = jnp.maximum(
            jnp.dot(xd_out, w1a_ref[...], preferred_element_type=jnp.float32)
            + jnp.dot(xt_out, w1b_ref[...], preferred_element_type=jnp.float32)
            + b1_ref[...], 0.0)
        z = jnp.maximum(
            jnp.dot(z, w2_ref[...], preferred_element_type=jnp.float32)
            + b2_ref[...], 0.0)
        o_ref[...] = (jnp.dot(z, w3_ref[...],
                              preferred_element_type=jnp.float32)
                      + b3_ref[...])

    return pl.pallas_call(
        body,
        out_shape=jax.ShapeDtypeStruct((bsz, 1), jnp.float32),
    )(emb_r, gflat, cbexp, fco, xd_b, pooled, xt_w, xt_b,
      w1a, w1b, b1, w2, b2, w3, b3)


def kernel(xd, xt, xt_edge_index, xt_batch, y, params):
    n = xt.shape[0]
    e = xt_edge_index.shape[1]
    bsz = xd.shape[0]
    n2 = n // _NC
    r = 2000
    inv = np.float32(1.0 / np.sqrt(1.0 + 1e-5))

    ce = _NS * _PAIR
    ep = -(-e // ce) * ce
    eps = ep // _NS
    nch = eps // _PAIR
    src = xt_edge_index[0]
    dst = xt_edge_index[1]
    src2 = jnp.pad(src, (0, ep - e)).astype(jnp.int32)
    dst2 = jnp.pad(dst, (0, ep - e), constant_values=n).astype(jnp.int32)

    prep = _make_sc_prep(n, ep, eps, nch, n2)
    locidx = prep(dst2)

    gin = params["gin"]
    bn = params["bn"]
    b1s = [gp["b1"].reshape(1, _DIM) for gp in gin]
    b2s = [gp["b2"].reshape(1, _DIM) for gp in gin]
    w2s = [gp["w2"] for gp in gin]
    scales = [(bp["g"] * inv).reshape(1, _DIM) for bp in bn]
    shifts = [bp["b"].reshape(1, _DIM) for bp in bn]

    fco = params["fc1_xd_w"]
    fc3 = fco.reshape(32, 121, 128)
    fcstack = jnp.concatenate(
        [jnp.pad(fc3, ((0, 0), (k, 7 - k), (0, 0))) for k in range(8)],
        axis=0).reshape(256, 128 * 128)
    w2d = jnp.transpose(params["conv_w"], (1, 2, 0)).reshape(100, 256)
    cbexp = jnp.repeat(params["conv_b"], 121).reshape(1, 3872)
    xdf = xd.reshape(bsz * 100, 1).astype(jnp.int32)
    batch3 = xt_batch.reshape(n // r, 1, r).astype(jnp.int32)
    cls_w1 = params["cls_w1"]

    gout, emb = _tc_build_g(w2d, fcstack, xdf, params["emb_xd"], bsz)
    gflat = gout.reshape(12800, 128)
    emb_r = emb.reshape(bsz, 12800)

    sc_agg = _make_sc_agg(n, ep, eps, nch, n2)
    g = _tc_g0(xt, gin[0]["w1"], n, r)
    for l in range(5):
        agg = sc_agg(g, src2, locidx)
        if l < 4:
            g = _tc_layer(g, agg, b1s[l], w2s[l], b2s[l], scales[l],
                          shifts[l], gin[l + 1]["w1"], n, r)
        else:
            pooled = _tc_last(g, agg, batch3, b1s[l], w2s[l], b2s[l],
                              scales[l], shifts[l], bsz, n, r)

    out2 = _tc_head(
        emb_r, gflat, cbexp, fco, params["fc1_xd_b"].reshape(1, 128),
        pooled, params["fc1_xt_w"], params["fc1_xt_b"].reshape(1, 128),
        cls_w1[:128], cls_w1[128:], params["cls_b1"].reshape(1, 1024),
        params["cls_w2"], params["cls_b2"].reshape(1, 256),
        params["cls_w3"], params["cls_b3"].reshape(1, 1), bsz)
    return (out2.reshape(bsz), y)
